# baseline (speedup 1.0000x reference)
import functools

import jax
import jax.numpy as jnp
from jax import lax
from jax.experimental import pallas as pl
from jax.experimental.pallas import tpu as pltpu
from jax.experimental.pallas import tpu_sc as plsc

N_NODES = 100000
N_EDGES = 6400000
NC = 2
NS = 16
NW = NC * NS
E_PER_W = N_EDGES // NW
CHUNK = 2000
N_CHUNKS = E_PER_W // CHUNK
NBUF = 2
PAIRS = N_CHUNKS // NBUF
UNROLL = 5

_MESH = plsc.VectorSubcoreMesh(core_axis_name="c", subcore_axis_name="s")


def _finish(ss):
    ss = jnp.maximum(ss, jnp.float32(1e-16))
    i = plsc.bitcast(ss, jnp.int32)
    i = jnp.int32(0x5F3759DF) - (i >> 1)
    y = plsc.bitcast(i, jnp.float32)
    h = jnp.float32(0.5) * ss
    for _ in range(2):
        y = y * (jnp.float32(1.5) - h * y * y)
    w = ss * y
    return jnp.maximum(w, jnp.float32(1e-8))


@functools.partial(
    pl.kernel,
    mesh=_MESH,
    out_type=(
        jax.ShapeDtypeStruct((N_EDGES,), jnp.float32),
        jax.ShapeDtypeStruct((2 * N_EDGES,), jnp.int32),
    ),
    compiler_params=pltpu.CompilerParams(needs_layout_passes=False),
    scratch_types=[
        pltpu.VMEM((N_NODES,), jnp.int32),
        pltpu.VMEM((CHUNK,), jnp.int32),
        pltpu.VMEM((CHUNK,), jnp.int32),
        pltpu.VMEM((CHUNK,), jnp.int32),
        pltpu.VMEM((CHUNK,), jnp.int32),
        pltpu.VMEM((CHUNK,), jnp.float32),
        pltpu.VMEM((CHUNK,), jnp.float32),
        pltpu.VMEM((CHUNK,), jnp.float32),
        pltpu.VMEM((CHUNK,), jnp.float32),
        pltpu.SemaphoreType.DMA((NBUF,)),
        pltpu.SemaphoreType.DMA((NBUF,)),
        pltpu.SemaphoreType.DMA,
    ],
)
def _distance_sc(tab_hbm, ei_hbm, out_hbm, eiout_hbm,
                 table, srcb0, srcb1, dstb0, dstb1, ainb0, ainb1,
                 aoutb0, aoutb1, insem, wbsem, echosem):
    wid = lax.axis_index("s") * NC + lax.axis_index("c")
    base = wid * E_PER_W
    echo_sz = 2 * N_EDGES // NW
    echo_off = wid * echo_sz
    pltpu.async_copy(ei_hbm.at[pl.ds(echo_off, echo_sz)],
                     eiout_hbm.at[pl.ds(echo_off, echo_sz)], echosem)
    srcb = [srcb0, srcb1]
    dstb = [dstb0, dstb1]
    ainb = [ainb0, ainb1]
    aoutb = [aoutb0, aoutb1]

    def issue_in(ci, b, c):
        off = base + ci * CHUNK
        pltpu.async_copy(ei_hbm.at[pl.ds(off, CHUNK)], srcb[b], insem.at[b])
        pltpu.async_copy(ei_hbm.at[pl.ds(N_EDGES + off, CHUNK)], dstb[b],
                         insem.at[b])
        if c > 0:
            pltpu.async_copy(out_hbm.at[pl.ds(off, CHUNK)], ainb[b],
                             insem.at[b])

    def wait_in(b, c):
        pltpu.make_async_copy(ei_hbm.at[pl.ds(0, CHUNK)], srcb[b],
                              insem.at[b]).wait()
        pltpu.make_async_copy(ei_hbm.at[pl.ds(0, CHUNK)], dstb[b],
                              insem.at[b]).wait()
        if c > 0:
            pltpu.make_async_copy(out_hbm.at[pl.ds(0, CHUNK)], ainb[b],
                                  insem.at[b]).wait()

    def issue_wb(ci, b):
        off = base + ci * CHUNK
        pltpu.async_copy(aoutb[b], out_hbm.at[pl.ds(off, CHUNK)],
                         wbsem.at[b])

    def wait_wb(b):
        pltpu.make_async_copy(aoutb[b], out_hbm.at[pl.ds(0, CHUNK)],
                              wbsem.at[b]).wait()

    hi_mask = jnp.int32(-65536)

    for c in range(2):
        pltpu.sync_copy(tab_hbm.at[pl.ds(c * N_NODES, N_NODES)], table)
        issue_in(0, 0, c)

        def pair_body(p, _, c=c):
            for b in range(NBUF):
                ci = p * NBUF + b

                @pl.when(ci + 1 < N_CHUNKS)
                def _prefetch():
                    issue_in(ci + 1, 1 - b, c)

                wait_in(b, c)

                @pl.when(ci >= NBUF)
                def _drain_prev_wb():
                    wait_wb(b)

                @plsc.parallel_loop(0, CHUNK, 16, unroll=UNROLL)
                def _group(j, b=b, c=c):
                    sl = pl.ds(j, 16)
                    s = srcb[b][sl]
                    d = dstb[b][sl]
                    gs = plsc.load_gather(table, [s])
                    gd = plsc.load_gather(table, [d])
                    if c == 0:
                        xs = plsc.bitcast(gs, jnp.float32)
                        xd = plsc.bitcast(gd, jnp.float32)
                        dx = xs - xd
                        aoutb[b][sl] = dx * dx
                    else:
                        ys = plsc.bitcast(gs & hi_mask, jnp.float32)
                        yd = plsc.bitcast(gd & hi_mask, jnp.float32)
                        zs = plsc.bitcast(gs << 16, jnp.float32)
                        zd = plsc.bitcast(gd << 16, jnp.float32)
                        dy = ys - yd
                        dz = zs - zd
                        ss = ainb[b][sl] + dy * dy + dz * dz
                        aoutb[b][sl] = _finish(ss)

                issue_wb(ci, b)
            return 0

        lax.fori_loop(0, PAIRS, pair_body, 0)
        wait_wb(0)
        wait_wb(1)

    pltpu.make_async_copy(ei_hbm.at[pl.ds(0, echo_sz)],
                          eiout_hbm.at[pl.ds(0, echo_sz)], echosem).wait()


def kernel(pos, edge_index):
    xbits = lax.bitcast_convert_type(pos[:, 0], jnp.int32)
    y16 = lax.bitcast_convert_type(
        pos[:, 1].astype(jnp.bfloat16), jnp.uint16).astype(jnp.uint32)
    z16 = lax.bitcast_convert_type(
        pos[:, 2].astype(jnp.bfloat16), jnp.uint16).astype(jnp.uint32)
    yz = lax.bitcast_convert_type((y16 << 16) | z16, jnp.int32)
    tab = jnp.concatenate([xbits, yz])
    ei_flat = edge_index.reshape(-1)
    w, ei_out = _distance_sc(tab, ei_flat)
    return ei_out.reshape(2, N_EDGES), w


# speedup vs baseline: 10.5645x; 10.5645x over previous
import functools

import jax
import jax.numpy as jnp
from jax import lax
from jax.experimental import pallas as pl
from jax.experimental.pallas import tpu as pltpu
from jax.experimental.pallas import tpu_sc as plsc

N_NODES = 100000
N_EDGES = 6400000
NC = 2
NS = 16
NW = NC * NS
E_PER_W = N_EDGES // NW
CHUNK = 2000
N_CHUNKS = E_PER_W // CHUNK
NBUF = 2
PAIRS = N_CHUNKS // NBUF
UNROLL = 5

_MESH = plsc.VectorSubcoreMesh(core_axis_name="c", subcore_axis_name="s")


def _finish(ss):
    ss = jnp.maximum(ss, jnp.float32(1e-16))
    i = plsc.bitcast(ss, jnp.int32)
    i = jnp.int32(0x5F3759DF) - (i >> 1)
    y = plsc.bitcast(i, jnp.float32)
    h = jnp.float32(0.5) * ss
    for _ in range(2):
        y = y * (jnp.float32(1.5) - h * y * y)
    w = ss * y
    return jnp.maximum(w, jnp.float32(1e-8))


@functools.partial(
    pl.kernel,
    mesh=_MESH,
    out_type=jax.ShapeDtypeStruct((N_EDGES,), jnp.float32),
    compiler_params=pltpu.CompilerParams(needs_layout_passes=False),
    scratch_types=[
        pltpu.VMEM((N_NODES,), jnp.int32),
        pltpu.VMEM((CHUNK,), jnp.int32),
        pltpu.VMEM((CHUNK,), jnp.int32),
        pltpu.VMEM((CHUNK,), jnp.int32),
        pltpu.VMEM((CHUNK,), jnp.int32),
        pltpu.VMEM((CHUNK,), jnp.float32),
        pltpu.VMEM((CHUNK,), jnp.float32),
        pltpu.VMEM((CHUNK,), jnp.float32),
        pltpu.VMEM((CHUNK,), jnp.float32),
        pltpu.SemaphoreType.DMA((NBUF,)),
        pltpu.SemaphoreType.DMA((NBUF,)),
    ],
)
def _distance_sc(tab_hbm, ei_hbm, out_hbm,
                 table, srcb0, srcb1, dstb0, dstb1, ainb0, ainb1,
                 aoutb0, aoutb1, insem, wbsem):
    wid = lax.axis_index("s") * NC + lax.axis_index("c")
    base = wid * E_PER_W
    srcb = [srcb0, srcb1]
    dstb = [dstb0, dstb1]
    ainb = [ainb0, ainb1]
    aoutb = [aoutb0, aoutb1]

    def issue_in(ci, b, c):
        off = base + ci * CHUNK
        pltpu.async_copy(ei_hbm.at[pl.ds(off, CHUNK)], srcb[b], insem.at[b])
        pltpu.async_copy(ei_hbm.at[pl.ds(N_EDGES + off, CHUNK)], dstb[b],
                         insem.at[b])
        if c > 0:
            pltpu.async_copy(out_hbm.at[pl.ds(off, CHUNK)], ainb[b],
                             insem.at[b])

    def wait_in(b, c):
        pltpu.make_async_copy(ei_hbm.at[pl.ds(0, CHUNK)], srcb[b],
                              insem.at[b]).wait()
        pltpu.make_async_copy(ei_hbm.at[pl.ds(0, CHUNK)], dstb[b],
                              insem.at[b]).wait()
        if c > 0:
            pltpu.make_async_copy(out_hbm.at[pl.ds(0, CHUNK)], ainb[b],
                                  insem.at[b]).wait()

    def issue_wb(ci, b):
        off = base + ci * CHUNK
        pltpu.async_copy(aoutb[b], out_hbm.at[pl.ds(off, CHUNK)],
                         wbsem.at[b])

    def wait_wb(b):
        pltpu.make_async_copy(aoutb[b], out_hbm.at[pl.ds(0, CHUNK)],
                              wbsem.at[b]).wait()

    hi_mask = jnp.int32(-65536)

    for c in range(2):
        pltpu.sync_copy(tab_hbm.at[pl.ds(c * N_NODES, N_NODES)], table)
        issue_in(0, 0, c)

        def pair_body(p, _, c=c):
            for b in range(NBUF):
                ci = p * NBUF + b

                @pl.when(ci + 1 < N_CHUNKS)
                def _prefetch():
                    issue_in(ci + 1, 1 - b, c)

                wait_in(b, c)

                @pl.when(ci >= NBUF)
                def _drain_prev_wb():
                    wait_wb(b)

                @plsc.parallel_loop(0, CHUNK, 16, unroll=UNROLL)
                def _group(j, b=b, c=c):
                    sl = pl.ds(j, 16)
                    s = srcb[b][sl]
                    d = dstb[b][sl]
                    gs = plsc.load_gather(table, [s])
                    gd = plsc.load_gather(table, [d])
                    if c == 0:
                        xs = plsc.bitcast(gs, jnp.float32)
                        xd = plsc.bitcast(gd, jnp.float32)
                        dx = xs - xd
                        aoutb[b][sl] = dx * dx
                    else:
                        ys = plsc.bitcast(gs & hi_mask, jnp.float32)
                        yd = plsc.bitcast(gd & hi_mask, jnp.float32)
                        zs = plsc.bitcast(gs << 16, jnp.float32)
                        zd = plsc.bitcast(gd << 16, jnp.float32)
                        dy = ys - yd
                        dz = zs - zd
                        ss = ainb[b][sl] + dy * dy + dz * dz
                        aoutb[b][sl] = _finish(ss)

                issue_wb(ci, b)
            return 0

        lax.fori_loop(0, PAIRS, pair_body, 0)
        wait_wb(0)
        wait_wb(1)


def kernel(pos, edge_index):
    xbits = lax.bitcast_convert_type(pos[:, 0], jnp.int32)
    y16 = lax.bitcast_convert_type(
        pos[:, 1].astype(jnp.bfloat16), jnp.uint16).astype(jnp.uint32)
    z16 = lax.bitcast_convert_type(
        pos[:, 2].astype(jnp.bfloat16), jnp.uint16).astype(jnp.uint32)
    yz = lax.bitcast_convert_type((y16 << 16) | z16, jnp.int32)
    tab = jnp.concatenate([xbits, yz])
    ei_flat = edge_index.reshape(-1)
    w = _distance_sc(tab, ei_flat)
    return edge_index, w
